# Initial kernel scaffold; baseline (speedup 1.0000x reference)
import jax
import jax.numpy as jnp
from jax.experimental import pallas as pl


def kernel(x, edge_index, W1, b1, W2, b2, Wh, bh):
    raise NotImplementedError("write your pallas kernel here")


# speedup vs baseline: 7.8648x; 7.8648x over previous
import functools

import jax
import jax.numpy as jnp
from jax import lax
from jax.experimental import pallas as pl
from jax.experimental.pallas import tpu as pltpu
from jax.experimental.pallas import tpu_sc as plsc

_CHUNK = 128
_NCORES = 2
_NSUB = 16
_NWORKERS = _NCORES * _NSUB


def _sc_degree(dst2d, iota, zeros, *, nch_w, npad):
    mesh = plsc.VectorSubcoreMesh(core_axis_name="c", subcore_axis_name="s")
    hrows = npad // _CHUNK

    @functools.partial(
        pl.kernel,
        out_type=jax.ShapeDtypeStruct((_NCORES, hrows, _CHUNK), jnp.float32),
        mesh=mesh,
        compiler_params=pltpu.CompilerParams(needs_layout_passes=False),
        scratch_types=[
            pltpu.VMEM((nch_w, _CHUNK), jnp.int32),
            pltpu.VMEM((hrows, _CHUNK), jnp.float32),
            pltpu.VMEM((hrows,), jnp.int32),
            pltpu.VMEM_SHARED((hrows, _CHUNK), jnp.float32),
        ],
    )
    def k(dst_hbm, iota_hbm, zeros_hbm, out_hbm, dst_v, hist, iota_v, acc):
        c = lax.axis_index("c")
        s = lax.axis_index("s")
        wid = s * _NCORES + c
        pltpu.sync_copy(dst_hbm.at[pl.ds(wid * nch_w, nch_w)], dst_v)
        pltpu.sync_copy(iota_hbm, iota_v)
        pltpu.sync_copy(zeros_hbm, hist)

        @pl.when(s == 0)
        def _():
            pltpu.sync_copy(zeros_hbm, acc)

        plsc.subcore_barrier()

        ones = jnp.ones((16,), jnp.float32)

        def body(i, carry):
            idx = dst_v[i >> 3, pl.ds((i & 7) * 16, 16)]
            plsc.addupdate_scatter(hist, [idx >> 7, idx & 127], ones)
            return carry

        lax.fori_loop(0, nch_w * 8, body, 0)
        pltpu.sync_copy(hist, acc.at[iota_v], add=True)
        plsc.subcore_barrier()

        @pl.when(s == 0)
        def _():
            pltpu.sync_copy(acc, out_hbm.at[c])

    return k(dst2d, iota, zeros)


def _sc_scatter(g, src2d, dst2d, zeros, *, nch_w, npad, rows_s, feat):
    mesh = plsc.VectorSubcoreMesh(core_axis_name="c", subcore_axis_name="s")

    @functools.partial(
        pl.kernel,
        out_type=jax.ShapeDtypeStruct((_NCORES, npad, feat), jnp.float32),
        mesh=mesh,
        scratch_types=[
            pltpu.VMEM((nch_w, _CHUNK), jnp.int32),
            pltpu.VMEM((nch_w, _CHUNK), jnp.int32),
            pltpu.VMEM((_CHUNK, feat), jnp.float32),
            pltpu.VMEM((_CHUNK, feat), jnp.float32),
            pltpu.VMEM_SHARED((npad, feat), jnp.float32),
            pltpu.SemaphoreType.DMA,
            pltpu.SemaphoreType.DMA,
        ],
    )
    def k(g_hbm, src_hbm, dst_hbm, zeros_hbm, out_hbm,
          src_v, dst_v, rows_a, rows_b, acc, sem_a, sem_b):
        c = lax.axis_index("c")
        s = lax.axis_index("s")
        wid = s * _NCORES + c
        pltpu.sync_copy(src_hbm.at[pl.ds(wid * nch_w, nch_w)], src_v)
        pltpu.sync_copy(dst_hbm.at[pl.ds(wid * nch_w, nch_w)], dst_v)
        pltpu.sync_copy(zeros_hbm.at[pl.ds(s * rows_s, rows_s)],
                        acc.at[pl.ds(s * rows_s, rows_s)])
        plsc.subcore_barrier()

        def body(i, carry):
            pltpu.async_copy(g_hbm.at[src_v.at[i]], rows_a, sem_a).wait()
            pltpu.sync_copy(rows_a, acc.at[dst_v.at[i]], add=True)
            return carry

        lax.fori_loop(0, nch_w, body, 0)
        plsc.subcore_barrier()
        pltpu.sync_copy(acc.at[pl.ds(s * rows_s, rows_s)],
                        out_hbm.at[c, pl.ds(s * rows_s, rows_s)])

    return k(g, src2d, dst2d, zeros)


def _dinv(d0_ref, d1_ref):
    deg = d0_ref[...] + d1_ref[...] + 1.0
    return lax.rsqrt(jnp.maximum(deg, 1.0))


def _tc_first(xpad, w1, d0, d1, *, npad, hid):
    def body(x_ref, w_ref, d0_ref, d1_ref, g_ref):
        h = jnp.dot(x_ref[...], w_ref[...], preferred_element_type=jnp.float32)
        g_ref[...] = _dinv(d0_ref, d1_ref) * h

    return pl.pallas_call(
        body, out_shape=jax.ShapeDtypeStruct((npad, hid), jnp.float32),
    )(xpad, w1, d0, d1)


def _tc_mid(p0, p1, g1, d0, d1, w2, b1, *, npad, hid):
    def body(p0_ref, p1_ref, g1_ref, d0_ref, d1_ref, w_ref, b_ref, g_ref):
        dinv = _dinv(d0_ref, d1_ref)
        agg = dinv * (p0_ref[...] + p1_ref[...] + g1_ref[...]) + b_ref[...]
        h = jnp.maximum(agg, 0.0)
        g_ref[...] = dinv * jnp.dot(h, w_ref[...],
                                    preferred_element_type=jnp.float32)

    return pl.pallas_call(
        body, out_shape=jax.ShapeDtypeStruct((npad, hid), jnp.float32),
    )(p0, p1, g1, d0, d1, w2, b1)


def _tc_last(p0, p1, g2, d0, d1, wh, b2, bh, *, npad, hid, ncls):
    def body(p0_ref, p1_ref, g2_ref, d0_ref, d1_ref, w_ref, b2_ref, bh_ref,
             s_ref, h_ref):
        dinv = _dinv(d0_ref, d1_ref)
        agg = dinv * (p0_ref[...] + p1_ref[...] + g2_ref[...]) + b2_ref[...]
        h = jnp.maximum(agg, 0.0)
        h_ref[...] = h
        s_ref[...] = jnp.dot(h, w_ref[...],
                             preferred_element_type=jnp.float32) + bh_ref[...]

    return pl.pallas_call(
        body,
        out_shape=[jax.ShapeDtypeStruct((npad, ncls), jnp.float32),
                   jax.ShapeDtypeStruct((npad, hid), jnp.float32)],
    )(p0, p1, g2, d0, d1, wh, b2, bh)


def kernel(x, edge_index, W1, b1, W2, b2, Wh, bh):
    n, f_in = x.shape
    hid = W1.shape[1]
    ncls = Wh.shape[1]
    e = edge_index.shape[1]

    npad = ((n + 1 + 2047) // 2048) * 2048
    rows_s = npad // _NSUB
    nch_w = -(-e // (_NWORKERS * _CHUNK))
    nch_w += nch_w % 2
    epad = _NWORKERS * nch_w * _CHUNK

    pad = jnp.full((epad - e,), n, dtype=edge_index.dtype)
    src2d = jnp.concatenate([edge_index[0], pad]).reshape(-1, _CHUNK)
    dst2d = jnp.concatenate([edge_index[1], pad]).reshape(-1, _CHUNK)
    xpad = jnp.pad(x, ((0, npad - n), (0, 0)))
    zeros = jnp.zeros((npad, hid), jnp.float32)
    zeros_deg = jnp.zeros((npad // _CHUNK, _CHUNK), jnp.float32)
    iota = jnp.arange(npad // _CHUNK, dtype=jnp.int32)

    degp = _sc_degree(dst2d, iota, zeros_deg, nch_w=nch_w, npad=npad)
    degp = degp.reshape(_NCORES, npad, 1)
    d0, d1 = degp[0], degp[1]

    g1 = _tc_first(xpad, W1, d0, d1, npad=npad, hid=hid)
    parts1 = _sc_scatter(g1, src2d, dst2d, zeros,
                         nch_w=nch_w, npad=npad, rows_s=rows_s, feat=hid)
    g2 = _tc_mid(parts1[0], parts1[1], g1, d0, d1, W2, b1,
                 npad=npad, hid=hid)
    parts2 = _sc_scatter(g2, src2d, dst2d, zeros,
                         nch_w=nch_w, npad=npad, rows_s=rows_s, feat=hid)
    scores, h = _tc_last(parts2[0], parts2[1], g2, d0, d1, Wh, b2, bh,
                         npad=npad, hid=hid, ncls=ncls)
    return (scores[:n], h[:n])
